# baseline (speedup 1.0000x reference)
import functools

import jax
import jax.numpy as jnp
from jax import lax
from jax.experimental import pallas as pl
from jax.experimental.pallas import tpu as pltpu
from jax.experimental.pallas import tpu_sc as plsc

VOCAB = 1000
DP = 1024
B = 1024
T = 50
TP = 56
N_TOK = B * T

NC = 2
NS = 16
NW = NC * NS
PPW = B // NW
TPW = N_TOK // NW
GPP = TP // 8
NG = PPW * GPP
SG = 80
NSG = TPW // SG
L = 16
NVR = TPW // L
NCV = VOCAB // L - 1


def _lse_body(table_ref, out_ref):
    t = table_ref[...]
    m = jnp.max(t, axis=1, keepdims=True)
    s = jnp.sum(jnp.exp(t - m), axis=1, keepdims=True)
    out_ref[...] = m + jnp.log(s)


def _row_lse(table):
    return pl.pallas_call(
        _lse_body,
        out_shape=jax.ShapeDtypeStruct((VOCAB, 1), jnp.float32),
    )(table)


def _sc_body(tabp_hbm, tflat_hbm, lse_hbm, idxp_hbm, idx_hbm, tgt_hbm, pt_hbm,
             logits_hbm, part_hbm,
             idxp_v, idx_v, tgt_v, flat_v, tl_v, lse_v, acc_v, p_v,
             a0, a1, b0, b1,
             sem_t, sem_l, g0, g1, s0, s1):
    wid = lax.axis_index("s") * NC + lax.axis_index("c")
    base = wid * TPW
    baseb = wid * PPW

    pltpu.sync_copy(idxp_hbm.at[pl.ds(wid * (PPW * TP), PPW * TP)], idxp_v)
    pltpu.sync_copy(idx_hbm.at[pl.ds(base, TPW)], idx_v)
    pltpu.sync_copy(tgt_hbm.at[pl.ds(base, TPW)], tgt_v)
    pltpu.sync_copy(pt_hbm, p_v)

    @pl.loop(0, NVR)
    def _flat(i):
        off = pl.multiple_of(i * L, L)
        flat_v[pl.ds(off, L)] = idx_v[pl.ds(off, L)] * VOCAB + tgt_v[pl.ds(off, L)]

    for k in range(NSG):
        pltpu.async_copy(tflat_hbm.at[flat_v.at[pl.ds(k * SG, SG)]],
                         tl_v.at[pl.ds(k * SG, SG)], sem_t)
        pltpu.async_copy(lse_hbm.at[idx_v.at[pl.ds(k * SG, SG)]],
                         lse_v.at[pl.ds(k * SG, SG)], sem_l)
    for k in range(NSG):
        pltpu.make_async_copy(tflat_hbm.at[flat_v.at[pl.ds(k * SG, SG)]],
                              tl_v.at[pl.ds(k * SG, SG)], sem_t).wait()
        pltpu.make_async_copy(lse_hbm.at[idx_v.at[pl.ds(k * SG, SG)]],
                              lse_v.at[pl.ds(k * SG, SG)], sem_l).wait()

    @pl.loop(0, NVR, init_carry=jnp.zeros((L,), jnp.float32))
    def _nll(i, acc):
        off = pl.multiple_of(i * L, L)
        return acc + (lse_v[pl.ds(off, L)] - tl_v[pl.ds(off, L)])

    acc_v[...] = _nll
    pltpu.sync_copy(acc_v, part_hbm.at[wid])

    def _start_gather(j, buf, sem):
        off = pl.multiple_of(j * 8, 8)
        pltpu.async_copy(tabp_hbm.at[idxp_v.at[pl.ds(off, 8)]], buf, sem)

    def _wait_gather(j, buf, sem):
        off = pl.multiple_of(j * 8, 8)
        pltpu.make_async_copy(tabp_hbm.at[idxp_v.at[pl.ds(off, 8)]],
                              buf, sem).wait()

    def _dst(j):
        p = p_v[pl.ds(pl.multiple_of(j * L, L), L)][0]
        rof = pl.multiple_of(j * 8 - p * TP, 8)
        return logits_hbm.at[baseb + p].at[pl.ds(rof, 8)]

    def _compact(src, dst):
        @pl.loop(0, NCV)
        def _cp(c):
            off = pl.multiple_of(c * L, L)
            for r in range(8):
                dst[r, pl.ds(off, L)] = src[r, pl.ds(off, L)]

        for r in range(8):
            dst[r, pl.ds(VOCAB - L, L)] = src[r, pl.ds(VOCAB - L, L)]

    _start_gather(0, a0, g0)
    _start_gather(1, a1, g1)

    @pl.loop(0, NG - 2, step=2)
    def _pipe(c):
        for bsel, bufa, bufb, g, s in ((0, a0, b0, g0, s0), (1, a1, b1, g1, s1)):
            j = c + bsel
            _wait_gather(j, bufa, g)
            _compact(bufa, bufb)
            pltpu.async_copy(bufb, _dst(j), s)
            pltpu.make_async_copy(bufb, _dst(j), s).wait()
            _start_gather(j + 2, bufa, g)

    for bsel, bufa, bufb, g in ((0, a0, b0, g0), (1, a1, b1, g1)):
        j = NG - 2 + bsel
        _wait_gather(j, bufa, g)
        _compact(bufa, bufb)
        pltpu.sync_copy(bufb, _dst(j))


@functools.partial(
    pl.kernel,
    out_type=[
        jax.ShapeDtypeStruct((B, T, VOCAB), jnp.float32),
        jax.ShapeDtypeStruct((NW, L), jnp.float32),
    ],
    mesh=plsc.VectorSubcoreMesh(core_axis_name="c", subcore_axis_name="s",
                                num_cores=NC, num_subcores=NS),
    scratch_types=[
        pltpu.VMEM((PPW * TP,), jnp.int32),
        pltpu.VMEM((TPW,), jnp.int32),
        pltpu.VMEM((TPW,), jnp.int32),
        pltpu.VMEM((TPW,), jnp.int32),
        pltpu.VMEM((TPW,), jnp.float32),
        pltpu.VMEM((TPW,), jnp.float32),
        pltpu.VMEM((L,), jnp.float32),
        pltpu.VMEM((NG * L,), jnp.int32),
        pltpu.VMEM((8, DP), jnp.float32),
        pltpu.VMEM((8, DP), jnp.float32),
        pltpu.VMEM((8, VOCAB), jnp.float32),
        pltpu.VMEM((8, VOCAB), jnp.float32),
        pltpu.SemaphoreType.DMA,
        pltpu.SemaphoreType.DMA,
        pltpu.SemaphoreType.DMA,
        pltpu.SemaphoreType.DMA,
        pltpu.SemaphoreType.DMA,
        pltpu.SemaphoreType.DMA,
    ],
)
def _sc_gather(tabp_hbm, tflat_hbm, lse_hbm, idxp_hbm, idx_hbm, tgt_hbm,
               pt_hbm, logits_hbm, part_hbm, *scratch):
    _sc_body(tabp_hbm, tflat_hbm, lse_hbm, idxp_hbm, idx_hbm, tgt_hbm, pt_hbm,
             logits_hbm, part_hbm, *scratch)


@jax.jit
def kernel(idx, targets, table):
    idx_f = idx.reshape(-1)
    tgt_f = targets.reshape(-1)
    idx_pad = jnp.pad(idx, ((0, 0), (0, TP - T))).reshape(-1)
    table_pad = jnp.pad(table, ((0, 0), (0, DP - VOCAB)))
    p_tab = jnp.repeat(jnp.arange(NG, dtype=jnp.int32) // GPP, L)
    row_lse = _row_lse(table).reshape(VOCAB)
    logits, partials = _sc_gather(table_pad, table.reshape(-1), row_lse,
                                  idx_pad, idx_f, tgt_f, p_tab)
    loss = jnp.sum(partials) / N_TOK
    return (logits, loss)


# speedup vs baseline: 1.1235x; 1.1235x over previous
import functools

import jax
import jax.numpy as jnp
from jax import lax
from jax.experimental import pallas as pl
from jax.experimental.pallas import tpu as pltpu
from jax.experimental.pallas import tpu_sc as plsc

VOCAB = 1000
DP = 1024
B = 1024
T = 50
TP = 56
N_TOK = B * T

NC = 2
NS = 16
NW = NC * NS
PPW = B // NW
TPW = N_TOK // NW
SG = 80
NSG = TPW // SG
GPP = TP // 8
NG = PPW * GPP
L = 16
NVR = TPW // L


def _lse_body(table_ref, out_ref):
    t = table_ref[...]
    m = jnp.max(t, axis=1, keepdims=True)
    s = jnp.sum(jnp.exp(t - m), axis=1, keepdims=True)
    out_ref[...] = m + jnp.log(s)


def _row_lse(table):
    return pl.pallas_call(
        _lse_body,
        out_shape=jax.ShapeDtypeStruct((VOCAB, 1), jnp.float32),
    )(table)


def _sc_body(tab_hbm, tflat_hbm, lse_hbm, idxp_hbm, idx_hbm, tgt_hbm, pt_hbm,
             logits_hbm, part_hbm,
             idxp_v, idx_v, tgt_v, flat_v, tl_v, lse_v, acc_v, p_v,
             a0, a1, a2, a3,
             sem_t, sem_l, g0, g1, g2, g3, s0, s1, s2, s3):
    wid = lax.axis_index("s") * NC + lax.axis_index("c")
    base = wid * TPW
    baseb = wid * PPW

    pltpu.sync_copy(idxp_hbm.at[pl.ds(wid * (PPW * TP), PPW * TP)], idxp_v)
    pltpu.sync_copy(idx_hbm.at[pl.ds(base, TPW)], idx_v)
    pltpu.sync_copy(tgt_hbm.at[pl.ds(base, TPW)], tgt_v)
    pltpu.sync_copy(pt_hbm, p_v)

    @pl.loop(0, NVR)
    def _flat(i):
        off = pl.multiple_of(i * L, L)
        flat_v[pl.ds(off, L)] = idx_v[pl.ds(off, L)] * VOCAB + tgt_v[pl.ds(off, L)]

    for k in range(NSG):
        pltpu.async_copy(tflat_hbm.at[flat_v.at[pl.ds(k * SG, SG)]],
                         tl_v.at[pl.ds(k * SG, SG)], sem_t)
        pltpu.async_copy(lse_hbm.at[idx_v.at[pl.ds(k * SG, SG)]],
                         lse_v.at[pl.ds(k * SG, SG)], sem_l)
    for k in range(NSG):
        pltpu.make_async_copy(tflat_hbm.at[flat_v.at[pl.ds(k * SG, SG)]],
                              tl_v.at[pl.ds(k * SG, SG)], sem_t).wait()
        pltpu.make_async_copy(lse_hbm.at[idx_v.at[pl.ds(k * SG, SG)]],
                              lse_v.at[pl.ds(k * SG, SG)], sem_l).wait()

    @pl.loop(0, NVR, init_carry=jnp.zeros((L,), jnp.float32))
    def _nll(i, acc):
        off = pl.multiple_of(i * L, L)
        return acc + (lse_v[pl.ds(off, L)] - tl_v[pl.ds(off, L)])

    acc_v[...] = _nll
    pltpu.sync_copy(acc_v, part_hbm.at[wid])

    def _src(j):
        off = pl.multiple_of(j * 8, 8)
        return tab_hbm.at[idxp_v.at[pl.ds(off, 8)]]

    def _dst(j):
        p = p_v[pl.ds(pl.multiple_of(j * L, L), L)][0]
        rof = pl.multiple_of(j * 8 - p * TP, 8)
        return logits_hbm.at[baseb + p].at[pl.ds(rof, 8)]

    bufs = (a0, a1, a2, a3)
    gsem = (g0, g1, g2, g3)
    ssem = (s0, s1, s2, s3)

    for b in range(4):
        pltpu.async_copy(_src(b), bufs[b], gsem[b])

    @pl.loop(0, NG - 4, step=4)
    def _pipe(c):
        for b in range(4):
            j = c + b
            pltpu.make_async_copy(_src(j), bufs[b], gsem[b]).wait()
            pltpu.async_copy(bufs[b], _dst(j), ssem[b])
        for b in range(4):
            j = c + b
            pltpu.make_async_copy(bufs[b], _dst(j), ssem[b]).wait()
            pltpu.async_copy(_src(j + 4), bufs[b], gsem[b])

    for b in range(4):
        j = NG - 4 + b
        pltpu.make_async_copy(_src(j), bufs[b], gsem[b]).wait()
        pltpu.async_copy(bufs[b], _dst(j), ssem[b])
    for b in range(4):
        j = NG - 4 + b
        pltpu.make_async_copy(bufs[b], _dst(j), ssem[b]).wait()


@functools.partial(
    pl.kernel,
    out_type=[
        jax.ShapeDtypeStruct((B, T, DP), jnp.float32),
        jax.ShapeDtypeStruct((NW, L), jnp.float32),
    ],
    mesh=plsc.VectorSubcoreMesh(core_axis_name="c", subcore_axis_name="s",
                                num_cores=NC, num_subcores=NS),
    scratch_types=[
        pltpu.VMEM((PPW * TP,), jnp.int32),
        pltpu.VMEM((TPW,), jnp.int32),
        pltpu.VMEM((TPW,), jnp.int32),
        pltpu.VMEM((TPW,), jnp.int32),
        pltpu.VMEM((TPW,), jnp.float32),
        pltpu.VMEM((TPW,), jnp.float32),
        pltpu.VMEM((L,), jnp.float32),
        pltpu.VMEM((NG * L,), jnp.int32),
        pltpu.VMEM((8, DP), jnp.float32),
        pltpu.VMEM((8, DP), jnp.float32),
        pltpu.VMEM((8, DP), jnp.float32),
        pltpu.VMEM((8, DP), jnp.float32),
        pltpu.SemaphoreType.DMA,
        pltpu.SemaphoreType.DMA,
        pltpu.SemaphoreType.DMA,
        pltpu.SemaphoreType.DMA,
        pltpu.SemaphoreType.DMA,
        pltpu.SemaphoreType.DMA,
        pltpu.SemaphoreType.DMA,
        pltpu.SemaphoreType.DMA,
        pltpu.SemaphoreType.DMA,
        pltpu.SemaphoreType.DMA,
    ],
)
def _sc_gather(tab_hbm, tflat_hbm, lse_hbm, idxp_hbm, idx_hbm, tgt_hbm,
               pt_hbm, logits_hbm, part_hbm, *scratch):
    _sc_body(tab_hbm, tflat_hbm, lse_hbm, idxp_hbm, idx_hbm, tgt_hbm, pt_hbm,
             logits_hbm, part_hbm, *scratch)


@jax.jit
def kernel(idx, targets, table):
    idx_f = idx.reshape(-1)
    tgt_f = targets.reshape(-1)
    idx_pad = jnp.pad(idx, ((0, 0), (0, TP - T))).reshape(-1)
    table_pad = jnp.pad(table, ((0, 0), (0, DP - VOCAB)))
    p_tab = jnp.repeat(jnp.arange(NG, dtype=jnp.int32) // GPP, L)
    row_lse = _row_lse(table).reshape(VOCAB)
    logits_pad, partials = _sc_gather(table_pad, table.reshape(-1), row_lse,
                                      idx_pad, idx_f, tgt_f, p_tab)
    logits = logits_pad[:, :, :VOCAB]
    loss = jnp.sum(partials) / N_TOK
    return (logits, loss)
